# baseline (speedup 1.0000x reference)
import functools

import jax
import jax.numpy as jnp
from jax import lax
from jax.experimental import pallas as pl
from jax.experimental.pallas import tpu as pltpu
from jax.experimental.pallas import tpu_sc as plsc

B = 16384
N_CAT = 26
N_NUM = 13
V = 100000
D = 32
H = 32
SEQ = N_CAT + N_NUM + 1
F = N_NUM * H
NP = N_CAT * D
FOUT = SEQ * D
NCLS = D + F

_NC = 2
_NS = 16
_NW = _NC * _NS
PPW = NP // _NW
RPW = NCLS // _NW
HALF = B // 2


def _make_sc_gather():
  mesh = plsc.VectorSubcoreMesh(core_axis_name="c", subcore_axis_name="s")

  @functools.partial(
      pl.kernel,
      mesh=mesh,
      out_type=(),
      compiler_params=pltpu.CompilerParams(
          use_tc_tiling_on_sc=True, needs_layout_passes=False),
      scratch_types=[
          pltpu.VMEM((V,), jnp.float32),
          pltpu.VMEM((B,), jnp.int32),
          pltpu.VMEM((HALF,), jnp.float32),
          pltpu.VMEM((FOUT,), jnp.float32),
          pltpu.SemaphoreType.DMA,
          pltpu.SemaphoreType.DMA,
      ],
  )
  def sc_kernel(tt2, idx_t, posf, out, plane_v, idx_v, out_v, pos_v,
                sem, wsem):
    wid = lax.axis_index("s") * _NC + lax.axis_index("c")
    base = wid * PPW

    pltpu.sync_copy(posf, pos_v)

    def plane_step(k, prev_c):
      p = base + k
      c = p // D

      stream = pltpu.async_copy(tt2.at[p], plane_v, sem)

      @pl.when(c != prev_c)
      def _():
        pltpu.sync_copy(idx_t.at[c], idx_v)

      stream.wait()
      pv = plsc.load_gather(pos_v, [jnp.full((16,), D + p, jnp.int32)])

      CH = B // 8
      handles = [None] * 8
      for i in range(8):
        if i >= 4:
          handles[i - 4].wait()
        bo = (i % 4) * CH

        def gstep(j, carry, _i=i, _bo=bo):
          ivs = []
          for u in range(8):
            off = j * 128 + u * 16
            ivs.append(idx_v[pl.ds(_i * CH + off, 16)])
          gs = [plsc.load_gather(plane_v, [iv]) for iv in ivs]
          for u in range(8):
            off = j * 128 + u * 16
            out_v[pl.ds(_bo + off, 16)] = gs[u] + pv
          return carry

        lax.fori_loop(0, CH // 128, gstep, 0)
        handles[i] = pltpu.async_copy(
            out_v.at[pl.ds(bo, CH)], out.at[D + p, pl.ds(i * CH, CH)], wsem)
      for i in range(4, 8):
        handles[i].wait()

      return c

    lax.fori_loop(0, PPW, plane_step, jnp.int32(-1))

  return sc_kernel


_sc_gather = _make_sc_gather()


def _make_sc_inject():
  mesh = plsc.VectorSubcoreMesh(core_axis_name="c", subcore_axis_name="s")

  @functools.partial(
      pl.kernel,
      mesh=mesh,
      out_type=(),
      compiler_params=pltpu.CompilerParams(
          use_tc_tiling_on_sc=True, needs_layout_passes=False),
      scratch_types=[
          pltpu.VMEM((B,), jnp.float32),
      ],
  )
  def sc_inject(ncls, out, buf_v):
    wid = lax.axis_index("s") * _NC + lax.axis_index("c")

    def copy_row(k, carry):
      src = wid * RPW + k
      dst = jnp.where(src < D, src, src + NP)
      pltpu.sync_copy(ncls.at[src], buf_v)
      pltpu.sync_copy(buf_v, out.at[dst])
      return carry

    lax.fori_loop(0, RPW, copy_row, 0)

  return sc_inject


_sc_inject = _make_sc_inject()

BT = 2048
GRID = B // BT


def _mlp_body(x_ref, nf_ref, mf_ref, st_ref, w1_ref, b1_ref, w2_ref, b2_ref,
              maskc_ref, nullc_ref, clsp_ref, posn_ref, out_ref):
  xr = jnp.dot(st_ref[...], x_ref[...], preferred_element_type=jnp.float32)
  h = jnp.maximum(xr * w1_ref[...] + b1_ref[...], 0.0)
  y = jnp.dot(w2_ref[...], h, preferred_element_type=jnp.float32) + b2_ref[...]
  nfr = jnp.dot(st_ref[...], nf_ref[...], preferred_element_type=jnp.float32)
  mfr = jnp.dot(st_ref[...], mf_ref[...], preferred_element_type=jnp.float32)
  num = nfr * nullc_ref[...] + (1.0 - nfr) * (
      mfr * maskc_ref[...] + (1.0 - mfr) * y)
  clsb = jnp.broadcast_to(clsp_ref[...], (D, BT))
  out_ref[...] = jnp.concatenate([clsb, num + posn_ref[...]], axis=0)


def _mlp(xt, nft, mft, st, w1c, b1c, w2t, b2c, maskc, nullc, clspc, posnc):
  return pl.pallas_call(
      _mlp_body,
      grid=(GRID,),
      in_specs=[
          pl.BlockSpec((N_NUM, BT), lambda i: (0, i)),
          pl.BlockSpec((N_NUM, BT), lambda i: (0, i)),
          pl.BlockSpec((N_NUM, BT), lambda i: (0, i)),
          pl.BlockSpec((F, N_NUM), lambda i: (0, 0)),
          pl.BlockSpec((F, 1), lambda i: (0, 0)),
          pl.BlockSpec((F, 1), lambda i: (0, 0)),
          pl.BlockSpec((F, F), lambda i: (0, 0)),
          pl.BlockSpec((F, 1), lambda i: (0, 0)),
          pl.BlockSpec((F, 1), lambda i: (0, 0)),
          pl.BlockSpec((F, 1), lambda i: (0, 0)),
          pl.BlockSpec((D, 1), lambda i: (0, 0)),
          pl.BlockSpec((F, 1), lambda i: (0, 0)),
      ],
      out_specs=pl.BlockSpec((NCLS, BT), lambda i: (0, i)),
      out_shape=jax.ShapeDtypeStruct((NCLS, B), jnp.float32),
  )(xt, nft, mft, st, w1c, b1c, w2t, b2c, maskc, nullc, clspc, posnc)


def kernel(cat_indices, num_values, mask_flags, null_flags, emb_tables, W1,
           b1, W2, b2, mask_emb, null_emb, cls_token, pos_table):
  tt2 = emb_tables.transpose(0, 2, 1).reshape(NP, V)
  idx_t = cat_indices.T.astype(jnp.int32)

  xt = num_values.T
  nft = null_flags.T.astype(jnp.float32)
  mft = mask_flags.T.astype(jnp.float32)
  st = jnp.repeat(jnp.eye(N_NUM, dtype=jnp.float32), H, axis=0)
  w1c = W1.reshape(F, 1)
  b1c = b1.reshape(F, 1)
  w2t = jnp.einsum("nm,nhd->mdnh", jnp.eye(N_NUM, dtype=W2.dtype),
                   W2).reshape(F, F)
  b2c = b2.reshape(F, 1)
  maskc = mask_emb.reshape(F, 1)
  nullc = null_emb.reshape(F, 1)
  posf = pos_table.reshape(FOUT)
  clspc = (cls_token + pos_table[0]).reshape(D, 1)
  posnc = posf[D + NP:].reshape(F, 1)

  ncls = _mlp(xt, nft, mft, st, w1c, b1c, w2t, b2c, maskc, nullc, clspc,
              posnc)
  out_ref = jax.new_ref(jnp.zeros((FOUT, B), jnp.float32))
  _sc_gather(tt2, idx_t, posf, out_ref)
  _sc_inject(ncls, out_ref)
  out_t = out_ref[...]
  return out_t.T.reshape(B, SEQ, D)


# speedup vs baseline: 1.0298x; 1.0298x over previous
import functools

import jax
import jax.numpy as jnp
from jax import lax
from jax.experimental import pallas as pl
from jax.experimental.pallas import tpu as pltpu
from jax.experimental.pallas import tpu_sc as plsc

B = 16384
N_CAT = 26
N_NUM = 13
V = 100000
D = 32
H = 32
SEQ = N_CAT + N_NUM + 1
F = N_NUM * H
NP = N_CAT * D
FOUT = SEQ * D
NCLS = D + F

_NC = 2
_NS = 16
_NW = _NC * _NS
PPW = NP // _NW
RPW = NCLS // _NW
HALF = B // 2


def _make_sc_gather():
  mesh = plsc.VectorSubcoreMesh(core_axis_name="c", subcore_axis_name="s")

  @functools.partial(
      pl.kernel,
      mesh=mesh,
      out_type=jax.ShapeDtypeStruct((FOUT, B), jnp.float32),
      compiler_params=pltpu.CompilerParams(
          use_tc_tiling_on_sc=True, needs_layout_passes=False),
      scratch_types=[
          pltpu.VMEM((V,), jnp.float32),
          pltpu.VMEM((B,), jnp.int32),
          pltpu.VMEM((HALF,), jnp.float32),
          pltpu.VMEM((FOUT,), jnp.float32),
          pltpu.SemaphoreType.DMA,
          pltpu.SemaphoreType.DMA,
      ],
  )
  def sc_kernel(tt2, idx_t, ncls, posf, out, plane_v, idx_v, out_v, pos_v,
                sem, wsem):
    wid = lax.axis_index("s") * _NC + lax.axis_index("c")
    base = wid * PPW

    pltpu.sync_copy(posf, pos_v)

    def plane_step(k, prev_c):
      p = base + k
      c = p // D

      stream = pltpu.async_copy(tt2.at[p], plane_v, sem)

      @pl.when(k < RPW)
      def _():
        src = wid * RPW + k
        dst = jnp.where(src < D, src, src + NP)
        for h in range(2):
          pltpu.sync_copy(ncls.at[src, pl.ds(h * HALF, HALF)], out_v)
          pltpu.sync_copy(out_v, out.at[dst, pl.ds(h * HALF, HALF)])

      @pl.when(c != prev_c)
      def _():
        pltpu.sync_copy(idx_t.at[c], idx_v)

      stream.wait()
      pv = plsc.load_gather(pos_v, [jnp.full((16,), D + p, jnp.int32)])

      CH = B // 8
      handles = [None] * 8
      for i in range(8):
        if i >= 4:
          handles[i - 4].wait()
        bo = (i % 4) * CH

        def gstep(j, carry, _i=i, _bo=bo):
          ivs = []
          for u in range(8):
            off = j * 128 + u * 16
            ivs.append(idx_v[pl.ds(_i * CH + off, 16)])
          gs = [plsc.load_gather(plane_v, [iv]) for iv in ivs]
          for u in range(8):
            off = j * 128 + u * 16
            out_v[pl.ds(_bo + off, 16)] = gs[u] + pv
          return carry

        lax.fori_loop(0, CH // 128, gstep, 0)
        handles[i] = pltpu.async_copy(
            out_v.at[pl.ds(bo, CH)], out.at[D + p, pl.ds(i * CH, CH)], wsem)
      for i in range(4, 8):
        handles[i].wait()

      return c

    lax.fori_loop(0, PPW, plane_step, jnp.int32(-1))

  return sc_kernel


_sc_gather = _make_sc_gather()

BT = 2048
GRID = B // BT


def _mlp_body(x_ref, nf_ref, mf_ref, st_ref, w1_ref, b1_ref, w2_ref, b2_ref,
              maskc_ref, nullc_ref, clsp_ref, posn_ref, out_ref):
  xr = jnp.dot(st_ref[...], x_ref[...], preferred_element_type=jnp.float32)
  h = jnp.maximum(xr * w1_ref[...] + b1_ref[...], 0.0)
  y = jnp.dot(w2_ref[...], h, preferred_element_type=jnp.float32) + b2_ref[...]
  nfr = jnp.dot(st_ref[...], nf_ref[...], preferred_element_type=jnp.float32)
  mfr = jnp.dot(st_ref[...], mf_ref[...], preferred_element_type=jnp.float32)
  num = nfr * nullc_ref[...] + (1.0 - nfr) * (
      mfr * maskc_ref[...] + (1.0 - mfr) * y)
  clsb = jnp.broadcast_to(clsp_ref[...], (D, BT))
  out_ref[...] = jnp.concatenate([clsb, num + posn_ref[...]], axis=0)


def _mlp(xt, nft, mft, st, w1c, b1c, w2t, b2c, maskc, nullc, clspc, posnc):
  return pl.pallas_call(
      _mlp_body,
      grid=(GRID,),
      in_specs=[
          pl.BlockSpec((N_NUM, BT), lambda i: (0, i)),
          pl.BlockSpec((N_NUM, BT), lambda i: (0, i)),
          pl.BlockSpec((N_NUM, BT), lambda i: (0, i)),
          pl.BlockSpec((F, N_NUM), lambda i: (0, 0)),
          pl.BlockSpec((F, 1), lambda i: (0, 0)),
          pl.BlockSpec((F, 1), lambda i: (0, 0)),
          pl.BlockSpec((F, F), lambda i: (0, 0)),
          pl.BlockSpec((F, 1), lambda i: (0, 0)),
          pl.BlockSpec((F, 1), lambda i: (0, 0)),
          pl.BlockSpec((F, 1), lambda i: (0, 0)),
          pl.BlockSpec((D, 1), lambda i: (0, 0)),
          pl.BlockSpec((F, 1), lambda i: (0, 0)),
      ],
      out_specs=pl.BlockSpec((NCLS, BT), lambda i: (0, i)),
      out_shape=jax.ShapeDtypeStruct((NCLS, B), jnp.float32),
  )(xt, nft, mft, st, w1c, b1c, w2t, b2c, maskc, nullc, clspc, posnc)


def kernel(cat_indices, num_values, mask_flags, null_flags, emb_tables, W1,
           b1, W2, b2, mask_emb, null_emb, cls_token, pos_table):
  tt2 = emb_tables.transpose(0, 2, 1).reshape(NP, V)
  idx_t = cat_indices.T.astype(jnp.int32)

  xt = num_values.T
  nft = null_flags.T.astype(jnp.float32)
  mft = mask_flags.T.astype(jnp.float32)
  st = jnp.repeat(jnp.eye(N_NUM, dtype=jnp.float32), H, axis=0)
  w1c = W1.reshape(F, 1)
  b1c = b1.reshape(F, 1)
  w2t = jnp.einsum("nm,nhd->mdnh", jnp.eye(N_NUM, dtype=W2.dtype),
                   W2).reshape(F, F)
  b2c = b2.reshape(F, 1)
  maskc = mask_emb.reshape(F, 1)
  nullc = null_emb.reshape(F, 1)
  posf = pos_table.reshape(FOUT)
  clspc = (cls_token + pos_table[0]).reshape(D, 1)
  posnc = posf[D + NP:].reshape(F, 1)

  ncls = _mlp(xt, nft, mft, st, w1c, b1c, w2t, b2c, maskc, nullc, clspc,
              posnc)
  out_t = _sc_gather(tt2, idx_t, ncls, posf)
  return out_t.T.reshape(B, SEQ, D)
